# Initial kernel scaffold; baseline (speedup 1.0000x reference)
import jax
import jax.numpy as jnp
from jax.experimental import pallas as pl


def kernel(node_states, rel0, rel1, W0a, b0a, W0b, b0b, W1a, b1a, W1b, b1b, Wu1, bu1, Wu2, bu2):
    raise NotImplementedError("write your pallas kernel here")


# speedup vs baseline: 3.0957x; 3.0957x over previous
import jax
import jax.numpy as jnp
from jax import lax
from jax.experimental import pallas as pl
from jax.experimental.pallas import tpu as pltpu
from jax.experimental.pallas import tpu_sc as plsc

N = 10000
NPAD = 10240
H = 128
CH = 128
GRP = 8
NW = 32
NS = 16

_mesh = plsc.VectorSubcoreMesh(core_axis_name="c", subcore_axis_name="s")


def _worker_loop(w, idx_hbm, idxv, ngroups_total, nchunks_real, chunk_fn):
    nloc = -(-ngroups_total // NW)

    def it(g, carry):
        group = w * nloc + g

        @pl.when(group < ngroups_total)
        def _():
            pltpu.sync_copy(idx_hbm.at[pl.ds(group * GRP, GRP)], idxv)
            for jj in range(GRP):
                chunk = group * GRP + jj

                @pl.when(chunk < nchunks_real)
                def _():
                    chunk_fn(chunk, jj)

        return carry

    lax.fori_loop(0, nloc, it, 0)


def _sc_gather(table, idx0, idx1, c0_real, c1_real):
    def body(table_hbm, idx0_hbm, idx1_hbm, out0_hbm, out1_hbm,
             idxv, rowv, sem):
        w = lax.axis_index("s") * 2 + lax.axis_index("c")

        def make_fn(out_hbm):
            def fn(chunk, jj):
                pltpu.async_copy(table_hbm.at[idxv.at[jj]], rowv, sem).wait()
                pltpu.sync_copy(rowv, out_hbm.at[pl.ds(chunk * CH, CH)])
            return fn

        _worker_loop(w, idx0_hbm, idxv, idx0.shape[0] // GRP, c0_real,
                     make_fn(out0_hbm))
        _worker_loop(w, idx1_hbm, idxv, idx1.shape[0] // GRP, c1_real,
                     make_fn(out1_hbm))

    f = pl.kernel(
        body,
        out_type=(jax.ShapeDtypeStruct((c0_real * CH, H), jnp.float32),
                  jax.ShapeDtypeStruct((c1_real * CH, H), jnp.float32)),
        mesh=_mesh,
        scratch_types=[pltpu.VMEM((GRP, CH), jnp.int32),
                       pltpu.VMEM((CH, H), jnp.float32),
                       pltpu.SemaphoreType.DMA],
    )
    return f(table, idx0, idx1)


def _sc_scatter(rows0, idx0, rows1, idx1, zeros_hbm, c0_real, c1_real):
    slab = NPAD // NS

    def body(rows0_hbm, idx0_hbm, rows1_hbm, idx1_hbm, z_hbm, out_hbm,
             idxv, datav, acc):
        c = lax.axis_index("c")
        s = lax.axis_index("s")
        w = s * 2 + c
        pltpu.sync_copy(z_hbm.at[pl.ds(s * slab, slab)],
                        acc.at[pl.ds(s * slab, slab)])
        plsc.subcore_barrier()

        def make_fn(rows_hbm):
            def fn(chunk, jj):
                pltpu.sync_copy(rows_hbm.at[pl.ds(chunk * CH, CH)], datav)
                pltpu.sync_copy(datav, acc.at[idxv.at[jj]], add=True)
            return fn

        _worker_loop(w, idx0_hbm, idxv, idx0.shape[0] // GRP, c0_real,
                     make_fn(rows0_hbm))
        _worker_loop(w, idx1_hbm, idxv, idx1.shape[0] // GRP, c1_real,
                     make_fn(rows1_hbm))
        plsc.subcore_barrier()
        pltpu.sync_copy(acc.at[pl.ds(s * slab, slab)],
                        out_hbm.at[c, pl.ds(s * slab, slab)])

    f = pl.kernel(
        body,
        out_type=jax.ShapeDtypeStruct((2, NPAD, H), jnp.float32),
        mesh=_mesh,
        scratch_types=[pltpu.VMEM((GRP, CH), jnp.int32),
                       pltpu.VMEM((CH, H), jnp.float32),
                       pltpu.VMEM_SHARED((NPAD, H), jnp.float32)],
    )
    return f(rows0, idx0, rows1, idx1, zeros_hbm)


def _mlp_body(x_ref, wa_ref, ba_ref, wb_ref, bb_ref, o_ref):
    x = x_ref[...]
    h = lax.dot_general(x, wa_ref[...], (((1,), (1,)), ((), ())),
                        preferred_element_type=jnp.float32)
    h = jnp.maximum(h + ba_ref[...], 0.0)
    o = lax.dot_general(h, wb_ref[...], (((1,), (1,)), ((), ())),
                        preferred_element_type=jnp.float32)
    o_ref[...] = o + bb_ref[...]


def _tc_mlp(x, wa, ba, wb, bb, bm):
    m, k = x.shape
    ko = wb.shape[0]
    return pl.pallas_call(
        _mlp_body,
        grid=(m // bm,),
        in_specs=[
            pl.BlockSpec((bm, k), lambda i: (i, 0)),
            pl.BlockSpec(wa.shape, lambda i: (0, 0)),
            pl.BlockSpec((1, ba.shape[0]), lambda i: (0, 0)),
            pl.BlockSpec(wb.shape, lambda i: (0, 0)),
            pl.BlockSpec((1, bb.shape[0]), lambda i: (0, 0)),
        ],
        out_specs=pl.BlockSpec((bm, ko), lambda i: (i, 0)),
        out_shape=jax.ShapeDtypeStruct((m, ko), jnp.float32),
    )(x, wa, ba.reshape(1, -1), wb, bb.reshape(1, -1))


def _update_body(p0_ref, p1_ref, ns_ref, wu1_ref, bu1_ref, wu2_ref, bu2_ref,
                 o_ref):
    sm = p0_ref[...] + p1_ref[...]
    ns = ns_ref[...]
    wu1 = wu1_ref[...]
    h = lax.dot_general(sm, wu1[:, :H], (((1,), (1,)), ((), ())),
                        preferred_element_type=jnp.float32)
    h = h + lax.dot_general(ns, wu1[:, H:], (((1,), (1,)), ((), ())),
                            preferred_element_type=jnp.float32)
    h = jnp.maximum(h + bu1_ref[...], 0.0)
    o = lax.dot_general(h, wu2_ref[...], (((1,), (1,)), ((), ())),
                        preferred_element_type=jnp.float32)
    o_ref[...] = o + bu2_ref[...]


def _tc_update(p0, p1, ns, wu1, bu1, wu2, bu2):
    bm = 1000
    return pl.pallas_call(
        _update_body,
        grid=(N // bm,),
        in_specs=[
            pl.BlockSpec((bm, H), lambda i: (i, 0)),
            pl.BlockSpec((bm, H), lambda i: (i, 0)),
            pl.BlockSpec((bm, H), lambda i: (i, 0)),
            pl.BlockSpec(wu1.shape, lambda i: (0, 0)),
            pl.BlockSpec((1, 2 * H), lambda i: (0, 0)),
            pl.BlockSpec(wu2.shape, lambda i: (0, 0)),
            pl.BlockSpec((1, H), lambda i: (0, 0)),
        ],
        out_specs=pl.BlockSpec((bm, H), lambda i: (i, 0)),
        out_shape=jax.ShapeDtypeStruct((N, H), jnp.float32),
    )(p0, p1, ns, wu1, bu1.reshape(1, -1), wu2, bu2.reshape(1, -1))


def kernel(node_states, rel0, rel1, W0a, b0a, W0b, b0b, W1a, b1a, W1b, b1b,
           Wu1, bu1, Wu2, bu2):
    idx0 = rel0.astype(jnp.int32).reshape(-1, CH)
    idx1 = rel1.astype(jnp.int32).reshape(-1, CH)
    c0_real = idx0.shape[0]
    c1_real = idx1.shape[0]
    pad0 = (-c0_real) % GRP
    pad1 = (-c1_real) % GRP
    if pad0:
        idx0 = jnp.concatenate([idx0, jnp.zeros((pad0, CH), jnp.int32)])
    if pad1:
        idx1 = jnp.concatenate([idx1, jnp.zeros((pad1, CH), jnp.int32)])

    g0, g1 = _sc_gather(node_states, idx0, idx1, c0_real, c1_real)

    inp0 = g0.reshape(-1, 2 * H)
    out0 = _tc_mlp(inp0, W0a, b0a, W0b, b0b, bm=2560)
    out1 = _tc_mlp(g1, W1a, b1a, W1b, b1b, bm=3200)

    rows0 = out0.reshape(-1, H)
    zeros = jnp.zeros((NPAD, H), jnp.float32)
    partials = _sc_scatter(rows0, idx0, out1, idx1, zeros, c0_real, c1_real)

    return _tc_update(partials[0, :N], partials[1, :N], node_states,
                      Wu1, bu1, Wu2, bu2)


# speedup vs baseline: 3.5761x; 1.1552x over previous
import jax
import jax.numpy as jnp
from jax import lax
from jax.experimental import pallas as pl
from jax.experimental.pallas import tpu as pltpu
from jax.experimental.pallas import tpu_sc as plsc

N = 10000
NPAD = 10240
H = 128
CH = 128
NW = 32
NS = 16

_mesh = plsc.VectorSubcoreMesh(core_axis_name="c", subcore_axis_name="s")


def _pad_chunks(idx, mult):
    pad = (-idx.shape[0]) % mult
    if pad:
        idx = jnp.concatenate([idx, jnp.zeros((pad, CH), jnp.int32)])
    return idx


def _sc_gather(table, idx0, c0_real):
    nloc = idx0.shape[0] // NW

    def body(table_hbm, idx_hbm, out_hbm, idxv, rowv, sem):
        w = lax.axis_index("s") * 2 + lax.axis_index("c")
        pltpu.sync_copy(idx_hbm.at[pl.ds(w * nloc, nloc)], idxv)

        def it(j, carry):
            chunk = w * nloc + j

            @pl.when(chunk < c0_real)
            def _():
                pltpu.async_copy(table_hbm.at[idxv.at[j]], rowv, sem).wait()
                pltpu.sync_copy(rowv, out_hbm.at[pl.ds(chunk * CH, CH)])

            return carry

        lax.fori_loop(0, nloc, it, 0)

    f = pl.kernel(
        body,
        out_type=jax.ShapeDtypeStruct((c0_real * CH, H), jnp.float32),
        mesh=_mesh,
        scratch_types=[pltpu.VMEM((nloc, CH), jnp.int32),
                       pltpu.VMEM((CH, H), jnp.float32),
                       pltpu.SemaphoreType.DMA],
    )
    return f(table, idx0)


def _sc_scatter(rows0, idx0, idx1, zeros_hbm, zc_hbm, c0_real, c1_real):
    n0 = idx0.shape[0] // NW
    n1 = idx1.shape[0] // NW
    slab = NPAD // NS
    cslab = NPAD // NS

    def body(rows0_hbm, idx0_hbm, idx1_hbm, z_hbm, zc, out_hbm, cnt_hbm,
             idxv0, idxv1, datav, onesv, acc, acc_cnt):
        c = lax.axis_index("c")
        s = lax.axis_index("s")
        w = s * 2 + c
        pltpu.sync_copy(z_hbm.at[pl.ds(s * slab, slab)],
                        acc.at[pl.ds(s * slab, slab)])
        pltpu.sync_copy(zc.at[pl.ds(s * cslab, cslab)],
                        acc_cnt.at[pl.ds(s * cslab, cslab)])
        ones = jnp.ones((16,), jnp.int32)
        for k in range(CH // 16):
            onesv[pl.ds(k * 16, 16)] = ones
        plsc.subcore_barrier()

        pltpu.sync_copy(idx0_hbm.at[pl.ds(w * n0, n0)], idxv0)
        pltpu.sync_copy(idx1_hbm.at[pl.ds(w * n1, n1)], idxv1)

        def it0(j, carry):
            chunk = w * n0 + j

            @pl.when(chunk < c0_real)
            def _():
                pltpu.sync_copy(rows0_hbm.at[pl.ds(chunk * CH, CH)], datav)
                pltpu.sync_copy(datav, acc.at[idxv0.at[j]], add=True)

            return carry

        lax.fori_loop(0, n0, it0, 0)

        def it1(j, carry):
            chunk = w * n1 + j

            @pl.when(chunk < c1_real)
            def _():
                pltpu.sync_copy(onesv, acc_cnt.at[idxv1.at[j]], add=True)

            return carry

        lax.fori_loop(0, n1, it1, 0)

        plsc.subcore_barrier()
        pltpu.sync_copy(acc.at[pl.ds(s * slab, slab)],
                        out_hbm.at[c, pl.ds(s * slab, slab)])
        pltpu.sync_copy(acc_cnt.at[pl.ds(s * cslab, cslab)],
                        cnt_hbm.at[c, pl.ds(s * cslab, cslab)])

    f = pl.kernel(
        body,
        out_type=(jax.ShapeDtypeStruct((2, NPAD, H), jnp.float32),
                  jax.ShapeDtypeStruct((2, NPAD), jnp.int32)),
        mesh=_mesh,
        scratch_types=[pltpu.VMEM((n0, CH), jnp.int32),
                       pltpu.VMEM((n1, CH), jnp.int32),
                       pltpu.VMEM((CH, H), jnp.float32),
                       pltpu.VMEM((CH,), jnp.int32),
                       pltpu.VMEM_SHARED((NPAD, H), jnp.float32),
                       pltpu.VMEM_SHARED((NPAD,), jnp.int32)],
    )
    return f(rows0, idx0, idx1, zeros_hbm, zc_hbm)


def _mlp_body(x_ref, wa_ref, ba_ref, wb_ref, bb_ref, o_ref):
    x = x_ref[...].astype(jnp.bfloat16)
    wa = wa_ref[...].astype(jnp.bfloat16)
    h = lax.dot_general(x, wa, (((1,), (1,)), ((), ())),
                        preferred_element_type=jnp.float32)
    h = jnp.maximum(h + ba_ref[...], 0.0).astype(jnp.bfloat16)
    wb = wb_ref[...].astype(jnp.bfloat16)
    o = lax.dot_general(h, wb, (((1,), (1,)), ((), ())),
                        preferred_element_type=jnp.float32)
    o_ref[...] = o + bb_ref[...]


def _tc_mlp(x, wa, ba, wb, bb, bm):
    m, k = x.shape
    ko = wb.shape[0]
    return pl.pallas_call(
        _mlp_body,
        grid=(m // bm,),
        in_specs=[
            pl.BlockSpec((bm, k), lambda i: (i, 0)),
            pl.BlockSpec(wa.shape, lambda i: (0, 0)),
            pl.BlockSpec((1, ba.shape[0]), lambda i: (0, 0)),
            pl.BlockSpec(wb.shape, lambda i: (0, 0)),
            pl.BlockSpec((1, bb.shape[0]), lambda i: (0, 0)),
        ],
        out_specs=pl.BlockSpec((bm, ko), lambda i: (i, 0)),
        out_shape=jax.ShapeDtypeStruct((m, ko), jnp.float32),
    )(x, wa, ba.reshape(1, -1), wb, bb.reshape(1, -1))


def _update_body(p0_ref, p1_ref, cnt_ref, ns_ref, w1a_ref, b1a_ref, w1b_ref,
                 b1b_ref, wu1_ref, bu1_ref, wu2_ref, bu2_ref, o_ref):
    ns = ns_ref[...]
    h1 = lax.dot_general(ns, w1a_ref[...], (((1,), (1,)), ((), ())),
                         preferred_element_type=jnp.float32)
    h1 = jnp.maximum(h1 + b1a_ref[...], 0.0)
    m1 = lax.dot_general(h1, w1b_ref[...], (((1,), (1,)), ((), ())),
                         preferred_element_type=jnp.float32)
    m1 = m1 + b1b_ref[...]
    sm = p0_ref[...] + p1_ref[...] + cnt_ref[...] * m1
    wu1 = wu1_ref[...]
    h = lax.dot_general(sm, wu1[:, :H], (((1,), (1,)), ((), ())),
                        preferred_element_type=jnp.float32)
    h = h + lax.dot_general(ns, wu1[:, H:], (((1,), (1,)), ((), ())),
                            preferred_element_type=jnp.float32)
    h = jnp.maximum(h + bu1_ref[...], 0.0)
    o = lax.dot_general(h, wu2_ref[...], (((1,), (1,)), ((), ())),
                        preferred_element_type=jnp.float32)
    o_ref[...] = o + bu2_ref[...]


def _tc_update(p0, p1, cnt, ns, w1a, b1a, w1b, b1b, wu1, bu1, wu2, bu2):
    bm = 1000
    row = lambda i: (i, 0)
    fix = lambda i: (0, 0)
    return pl.pallas_call(
        _update_body,
        grid=(N // bm,),
        in_specs=[
            pl.BlockSpec((bm, H), row),
            pl.BlockSpec((bm, H), row),
            pl.BlockSpec((bm, 1), row),
            pl.BlockSpec((bm, H), row),
            pl.BlockSpec(w1a.shape, fix),
            pl.BlockSpec((1, H), fix),
            pl.BlockSpec(w1b.shape, fix),
            pl.BlockSpec((1, H), fix),
            pl.BlockSpec(wu1.shape, fix),
            pl.BlockSpec((1, 2 * H), fix),
            pl.BlockSpec(wu2.shape, fix),
            pl.BlockSpec((1, H), fix),
        ],
        out_specs=pl.BlockSpec((bm, H), row),
        out_shape=jax.ShapeDtypeStruct((N, H), jnp.float32),
    )(p0, p1, cnt, ns, w1a, b1a.reshape(1, -1), w1b, b1b.reshape(1, -1),
      wu1, bu1.reshape(1, -1), wu2, bu2.reshape(1, -1))


def kernel(node_states, rel0, rel1, W0a, b0a, W0b, b0b, W1a, b1a, W1b, b1b,
           Wu1, bu1, Wu2, bu2):
    idx0 = rel0.astype(jnp.int32).reshape(-1, CH)
    idx1 = rel1.astype(jnp.int32).reshape(-1, CH)
    c0_real = idx0.shape[0]
    c1_real = idx1.shape[0]
    idx0 = _pad_chunks(idx0, NW * 8)
    idx1 = _pad_chunks(idx1, NW * 8)

    g0 = _sc_gather(node_states, idx0, c0_real)

    inp0 = g0.reshape(-1, 2 * H)
    out0 = _tc_mlp(inp0, W0a, b0a, W0b, b0b, bm=2560)

    rows0 = out0.reshape(-1, H)
    zeros = jnp.zeros((NPAD, H), jnp.float32)
    zcnt = jnp.zeros((NPAD,), jnp.int32)
    partials, cnts = _sc_scatter(rows0, idx0, idx1, zeros, zcnt,
                                 c0_real, c1_real)

    cnt = (cnts[0, :N] + cnts[1, :N]).astype(jnp.float32).reshape(N, 1)
    return _tc_update(partials[0, :N], partials[1, :N], cnt, node_states,
                      W1a, b1a, W1b, b1b, Wu1, bu1, Wu2, bu2)


# speedup vs baseline: 4.5279x; 1.2661x over previous
import jax
import jax.numpy as jnp
from jax import lax
from jax.experimental import pallas as pl
from jax.experimental.pallas import tpu as pltpu
from jax.experimental.pallas import tpu_sc as plsc

N = 10000
NPAD = 10240
H = 128
CH = 128
NW = 32
NS = 16

_mesh = plsc.VectorSubcoreMesh(core_axis_name="c", subcore_axis_name="s")


def _pad_chunks(idx, mult):
    pad = (-idx.shape[0]) % mult
    if pad:
        idx = jnp.concatenate([idx, jnp.zeros((pad, CH), jnp.int32)])
    return idx


def _sc_gather(table, idxa, idxb, c0_real):
    nloc = idxa.shape[0] // NW

    def body(table_hbm, idxa_hbm, idxb_hbm, out_hbm, idxva, idxvb, rowv, sem):
        w = lax.axis_index("s") * 2 + lax.axis_index("c")
        pltpu.sync_copy(idxa_hbm.at[pl.ds(w * nloc, nloc)], idxva)
        pltpu.sync_copy(idxb_hbm.at[pl.ds(w * nloc, nloc)], idxvb)

        def it(j, carry):
            chunk = w * nloc + j

            @pl.when(chunk < c0_real)
            def _():
                pltpu.async_copy(table_hbm.at[idxva.at[j]], rowv, sem).wait()
                pltpu.sync_copy(
                    rowv, out_hbm.at[pl.ds(chunk * CH, CH), pl.ds(0, H)])
                pltpu.async_copy(table_hbm.at[idxvb.at[j]], rowv, sem).wait()
                pltpu.sync_copy(
                    rowv, out_hbm.at[pl.ds(chunk * CH, CH), pl.ds(H, H)])

            return carry

        lax.fori_loop(0, nloc, it, 0)

    f = pl.kernel(
        body,
        out_type=jax.ShapeDtypeStruct((c0_real * CH, 2 * H), jnp.float32),
        mesh=_mesh,
        scratch_types=[pltpu.VMEM((nloc, CH), jnp.int32),
                       pltpu.VMEM((nloc, CH), jnp.int32),
                       pltpu.VMEM((CH, H), jnp.float32),
                       pltpu.SemaphoreType.DMA],
    )
    return f(table, idxa, idxb)


def _sc_scatter(rows0, idxa, idxb, idx1, zeros_hbm, zc_hbm, c0_real, c1_real):
    n0 = idxa.shape[0] // NW
    n1 = idx1.shape[0] // NW
    slab = NPAD // NS
    cslab = NPAD // NS

    def body(rows0_hbm, idxa_hbm, idxb_hbm, idx1_hbm, z_hbm, zc, out_hbm,
             cnt_hbm, idxva, idxvb, idxv1, datav, onesv, acc, acc_cnt):
        c = lax.axis_index("c")
        s = lax.axis_index("s")
        w = s * 2 + c
        pltpu.sync_copy(z_hbm.at[pl.ds(s * slab, slab)],
                        acc.at[pl.ds(s * slab, slab)])
        pltpu.sync_copy(zc.at[pl.ds(s * cslab, cslab)],
                        acc_cnt.at[pl.ds(s * cslab, cslab)])
        ones = jnp.ones((16,), jnp.int32)
        for k in range(CH // 16):
            onesv[pl.ds(k * 16, 16)] = ones
        plsc.subcore_barrier()

        pltpu.sync_copy(idxa_hbm.at[pl.ds(w * n0, n0)], idxva)
        pltpu.sync_copy(idxb_hbm.at[pl.ds(w * n0, n0)], idxvb)
        pltpu.sync_copy(idx1_hbm.at[pl.ds(w * n1, n1)], idxv1)

        def it0(j, carry):
            chunk = w * n0 + j

            @pl.when(chunk < c0_real)
            def _():
                pltpu.sync_copy(
                    rows0_hbm.at[pl.ds(chunk * CH, CH), pl.ds(0, H)], datav)
                pltpu.sync_copy(datav, acc.at[idxva.at[j]], add=True)
                pltpu.sync_copy(
                    rows0_hbm.at[pl.ds(chunk * CH, CH), pl.ds(H, H)], datav)
                pltpu.sync_copy(datav, acc.at[idxvb.at[j]], add=True)

            return carry

        lax.fori_loop(0, n0, it0, 0)

        def it1(j, carry):
            chunk = w * n1 + j

            @pl.when(chunk < c1_real)
            def _():
                pltpu.sync_copy(onesv, acc_cnt.at[idxv1.at[j]], add=True)

            return carry

        lax.fori_loop(0, n1, it1, 0)

        plsc.subcore_barrier()
        pltpu.sync_copy(acc.at[pl.ds(s * slab, slab)],
                        out_hbm.at[c, pl.ds(s * slab, slab)])
        pltpu.sync_copy(acc_cnt.at[pl.ds(s * cslab, cslab)],
                        cnt_hbm.at[c, pl.ds(s * cslab, cslab)])

    f = pl.kernel(
        body,
        out_type=(jax.ShapeDtypeStruct((2, NPAD, H), jnp.float32),
                  jax.ShapeDtypeStruct((2, NPAD), jnp.int32)),
        mesh=_mesh,
        scratch_types=[pltpu.VMEM((n0, CH), jnp.int32),
                       pltpu.VMEM((n0, CH), jnp.int32),
                       pltpu.VMEM((n1, CH), jnp.int32),
                       pltpu.VMEM((CH, H), jnp.float32),
                       pltpu.VMEM((CH,), jnp.int32),
                       pltpu.VMEM_SHARED((NPAD, H), jnp.float32),
                       pltpu.VMEM_SHARED((NPAD,), jnp.int32)],
    )
    return f(rows0, idxa, idxb, idx1, zeros_hbm, zc_hbm)


def _mlp_body(x_ref, wa_ref, ba_ref, wb_ref, bb_ref, o_ref):
    x = x_ref[...].astype(jnp.bfloat16)
    wa = wa_ref[...].astype(jnp.bfloat16)
    h = lax.dot_general(x, wa, (((1,), (1,)), ((), ())),
                        preferred_element_type=jnp.float32)
    h = jnp.maximum(h + ba_ref[...], 0.0).astype(jnp.bfloat16)
    wb = wb_ref[...].astype(jnp.bfloat16)
    o = lax.dot_general(h, wb, (((1,), (1,)), ((), ())),
                        preferred_element_type=jnp.float32)
    o_ref[...] = o + bb_ref[...]


def _tc_mlp(x, wa, ba, wb, bb, bm):
    m, k = x.shape
    ko = wb.shape[0]
    return pl.pallas_call(
        _mlp_body,
        grid=(m // bm,),
        in_specs=[
            pl.BlockSpec((bm, k), lambda i: (i, 0)),
            pl.BlockSpec(wa.shape, lambda i: (0, 0)),
            pl.BlockSpec((1, ba.shape[0]), lambda i: (0, 0)),
            pl.BlockSpec(wb.shape, lambda i: (0, 0)),
            pl.BlockSpec((1, bb.shape[0]), lambda i: (0, 0)),
        ],
        out_specs=pl.BlockSpec((bm, ko), lambda i: (i, 0)),
        out_shape=jax.ShapeDtypeStruct((m, ko), jnp.float32),
    )(x, wa, ba.reshape(1, -1), wb, bb.reshape(1, -1))


def _update_body(p0_ref, p1_ref, cnt_ref, ns_ref, w1a_ref, b1a_ref, w1b_ref,
                 b1b_ref, wu1_ref, bu1_ref, wu2_ref, bu2_ref, o_ref):
    ns = ns_ref[...]
    h1 = lax.dot_general(ns, w1a_ref[...], (((1,), (1,)), ((), ())),
                         preferred_element_type=jnp.float32)
    h1 = jnp.maximum(h1 + b1a_ref[...], 0.0)
    m1 = lax.dot_general(h1, w1b_ref[...], (((1,), (1,)), ((), ())),
                         preferred_element_type=jnp.float32)
    m1 = m1 + b1b_ref[...]
    sm = p0_ref[...] + p1_ref[...] + cnt_ref[...] * m1
    wu1 = wu1_ref[...]
    h = lax.dot_general(sm, wu1[:, :H], (((1,), (1,)), ((), ())),
                        preferred_element_type=jnp.float32)
    h = h + lax.dot_general(ns, wu1[:, H:], (((1,), (1,)), ((), ())),
                            preferred_element_type=jnp.float32)
    h = jnp.maximum(h + bu1_ref[...], 0.0)
    o = lax.dot_general(h, wu2_ref[...], (((1,), (1,)), ((), ())),
                        preferred_element_type=jnp.float32)
    o_ref[...] = o + bu2_ref[...]


def _tc_update(p0, p1, cnt, ns, w1a, b1a, w1b, b1b, wu1, bu1, wu2, bu2):
    bm = 1000
    row = lambda i: (i, 0)
    fix = lambda i: (0, 0)
    return pl.pallas_call(
        _update_body,
        grid=(N // bm,),
        in_specs=[
            pl.BlockSpec((bm, H), row),
            pl.BlockSpec((bm, H), row),
            pl.BlockSpec((bm, 1), row),
            pl.BlockSpec((bm, H), row),
            pl.BlockSpec(w1a.shape, fix),
            pl.BlockSpec((1, H), fix),
            pl.BlockSpec(w1b.shape, fix),
            pl.BlockSpec((1, H), fix),
            pl.BlockSpec(wu1.shape, fix),
            pl.BlockSpec((1, 2 * H), fix),
            pl.BlockSpec(wu2.shape, fix),
            pl.BlockSpec((1, H), fix),
        ],
        out_specs=pl.BlockSpec((bm, H), row),
        out_shape=jax.ShapeDtypeStruct((N, H), jnp.float32),
    )(p0, p1, cnt, ns, w1a, b1a.reshape(1, -1), w1b, b1b.reshape(1, -1),
      wu1, bu1.reshape(1, -1), wu2, bu2.reshape(1, -1))


def kernel(node_states, rel0, rel1, W0a, b0a, W0b, b0b, W1a, b1a, W1b, b1b,
           Wu1, bu1, Wu2, bu2):
    pairs = rel0.astype(jnp.int32).reshape(-1, 2)
    idxa = pairs[:, 0].reshape(-1, CH)
    idxb = pairs[:, 1].reshape(-1, CH)
    idx1 = rel1.astype(jnp.int32).reshape(-1, CH)
    c0_real = idxa.shape[0]
    c1_real = idx1.shape[0]
    idxa = _pad_chunks(idxa, NW * 8)
    idxb = _pad_chunks(idxb, NW * 8)
    idx1 = _pad_chunks(idx1, NW * 8)

    inp0 = _sc_gather(node_states, idxa, idxb, c0_real)
    out0 = _tc_mlp(inp0, W0a, b0a, W0b, b0b, bm=2560)

    zeros = jnp.zeros((NPAD, H), jnp.float32)
    zcnt = jnp.zeros((NPAD,), jnp.int32)
    partials, cnts = _sc_scatter(out0, idxa, idxb, idx1, zeros, zcnt,
                                 c0_real, c1_real)

    cnt = (cnts[0, :N] + cnts[1, :N]).astype(jnp.float32).reshape(N, 1)
    return _tc_update(partials[0, :N], partials[1, :N], cnt, node_states,
                      W1a, b1a, W1b, b1b, Wu1, bu1, Wu2, bu2)


# speedup vs baseline: 5.5404x; 1.2236x over previous
import jax
import jax.numpy as jnp
from jax import lax
from jax.experimental import pallas as pl
from jax.experimental.pallas import tpu as pltpu
from jax.experimental.pallas import tpu_sc as plsc

N = 10000
NPAD = 10240
H = 128
CH = 128
NW = 32
NS = 16

_mesh = plsc.VectorSubcoreMesh(core_axis_name="c", subcore_axis_name="s")


def _pad_chunks(idx, mult, pad_row):
    pad = (-idx.shape[0]) % mult
    if pad:
        idx = jnp.concatenate([idx, jnp.broadcast_to(pad_row, (pad, CH))])
    return idx


def _sc_gather(table, idxa, idxb):
    nloc = idxa.shape[0] // NW

    def body(table_hbm, idxa_hbm, idxb_hbm, out_hbm,
             idxva, idxvb, bufa, bufb, sa0, sa1, sb0, sb1):
        w = lax.axis_index("s") * 2 + lax.axis_index("c")
        pltpu.sync_copy(idxa_hbm.at[pl.ds(w * nloc, nloc)], idxva)
        pltpu.sync_copy(idxb_hbm.at[pl.ds(w * nloc, nloc)], idxvb)
        sems_a = (sa0, sa1)
        sems_b = (sb0, sb1)
        pend = [None, None]
        for j in range(nloc + 1):
            p = j & 1
            if j < nloc:
                da = pltpu.async_copy(table_hbm.at[idxva.at[j]],
                                      bufa.at[p], sems_a[p])
                db = pltpu.async_copy(table_hbm.at[idxvb.at[j]],
                                      bufb.at[p], sems_b[p])
            if j >= 1:
                q = (j - 1) & 1
                chunk = w * nloc + (j - 1)
                pa, pb = pend[q]
                pa.wait()
                pltpu.sync_copy(bufa.at[q],
                                out_hbm.at[pl.ds(chunk * CH, CH), pl.ds(0, H)])
                pb.wait()
                pltpu.sync_copy(bufb.at[q],
                                out_hbm.at[pl.ds(chunk * CH, CH), pl.ds(H, H)])
            if j < nloc:
                pend[p] = (da, db)

    f = pl.kernel(
        body,
        out_type=jax.ShapeDtypeStruct((idxa.shape[0] * CH, 2 * H),
                                      jnp.float32),
        mesh=_mesh,
        scratch_types=[pltpu.VMEM((nloc, CH), jnp.int32),
                       pltpu.VMEM((nloc, CH), jnp.int32),
                       pltpu.VMEM((2, CH, H), jnp.float32),
                       pltpu.VMEM((2, CH, H), jnp.float32),
                       pltpu.SemaphoreType.DMA,
                       pltpu.SemaphoreType.DMA,
                       pltpu.SemaphoreType.DMA,
                       pltpu.SemaphoreType.DMA],
    )
    return f(table, idxa, idxb)


def _sc_scatter(rows0, idxa, idxb, idx1, zeros_hbm, zc_hbm):
    n0 = idxa.shape[0] // NW
    n1 = idx1.shape[0] // NW
    slab = NPAD // NS

    def body(rows0_hbm, idxa_hbm, idxb_hbm, idx1_hbm, z_hbm, zc, out_hbm,
             cnt_hbm, ga, gb, idxv1, buf, onesv, acc, acc_cnt, s0, s1):
        c = lax.axis_index("c")
        s = lax.axis_index("s")
        w = s * 2 + c
        pltpu.sync_copy(z_hbm.at[pl.ds(s * slab, slab)],
                        acc.at[pl.ds(s * slab, slab)])
        pltpu.sync_copy(zc.at[pl.ds(s * slab, slab)],
                        acc_cnt.at[pl.ds(s * slab, slab)])
        ones = jnp.ones((16,), jnp.int32)
        for k in range(CH // 16):
            onesv[pl.ds(k * 16, 16)] = ones
        plsc.subcore_barrier()

        pltpu.sync_copy(idx1_hbm.at[pl.ds(w * n1, n1)], idxv1)

        sems = (s0, s1)
        T = 2 * n0
        pend = [None, None]
        for t in range(T + 1):
            p = t & 1
            if t < T:
                j, st = t >> 1, t & 1
                if st == 0 and j % 8 == 0:
                    g = j // 8
                    gp = g & 1
                    pltpu.sync_copy(
                        idxa_hbm.at[pl.ds(w * n0 + g * 8, 8)], ga.at[gp])
                    pltpu.sync_copy(
                        idxb_hbm.at[pl.ds(w * n0 + g * 8, 8)], gb.at[gp])
                chunk = w * n0 + j
                cols = pl.ds(0, H) if st == 0 else pl.ds(H, H)
                r = pltpu.async_copy(rows0_hbm.at[pl.ds(chunk * CH, CH), cols],
                                     buf.at[p], sems[p])
            if t >= 1:
                q = (t - 1) & 1
                j1, st1 = (t - 1) >> 1, (t - 1) & 1
                gp1 = (j1 // 8) & 1
                idxref = (ga if st1 == 0 else gb).at[gp1, j1 % 8]
                pend[q].wait()
                pltpu.sync_copy(buf.at[q], acc.at[idxref], add=True)
            if t < T:
                pend[p] = r

        def it1(j, carry):
            pltpu.sync_copy(onesv, acc_cnt.at[idxv1.at[j]], add=True)
            return carry

        lax.fori_loop(0, n1, it1, 0)

        plsc.subcore_barrier()
        pltpu.sync_copy(acc.at[pl.ds(s * slab, slab)],
                        out_hbm.at[c, pl.ds(s * slab, slab)])
        pltpu.sync_copy(acc_cnt.at[pl.ds(s * slab, slab)],
                        cnt_hbm.at[c, pl.ds(s * slab, slab)])

    f = pl.kernel(
        body,
        out_type=(jax.ShapeDtypeStruct((2, NPAD, H), jnp.float32),
                  jax.ShapeDtypeStruct((2, NPAD), jnp.int32)),
        mesh=_mesh,
        scratch_types=[pltpu.VMEM((2, 8, CH), jnp.int32),
                       pltpu.VMEM((2, 8, CH), jnp.int32),
                       pltpu.VMEM((n1, CH), jnp.int32),
                       pltpu.VMEM((2, CH, H), jnp.float32),
                       pltpu.VMEM((CH,), jnp.int32),
                       pltpu.VMEM_SHARED((NPAD, H), jnp.float32),
                       pltpu.VMEM_SHARED((NPAD,), jnp.int32),
                       pltpu.SemaphoreType.DMA,
                       pltpu.SemaphoreType.DMA],
    )
    return f(rows0, idxa, idxb, idx1, zeros_hbm, zc_hbm)


def _mlp_body(x_ref, wa_ref, ba_ref, wb_ref, bb_ref, o_ref):
    x = x_ref[...].astype(jnp.bfloat16)
    wa = wa_ref[...].astype(jnp.bfloat16)
    h = lax.dot_general(x, wa, (((1,), (1,)), ((), ())),
                        preferred_element_type=jnp.float32)
    h = jnp.maximum(h + ba_ref[...], 0.0).astype(jnp.bfloat16)
    wb = wb_ref[...].astype(jnp.bfloat16)
    o = lax.dot_general(h, wb, (((1,), (1,)), ((), ())),
                        preferred_element_type=jnp.float32)
    o_ref[...] = o + bb_ref[...]


def _tc_mlp(x, wa, ba, wb, bb, bm):
    m, k = x.shape
    ko = wb.shape[0]
    return pl.pallas_call(
        _mlp_body,
        grid=(m // bm,),
        in_specs=[
            pl.BlockSpec((bm, k), lambda i: (i, 0)),
            pl.BlockSpec(wa.shape, lambda i: (0, 0)),
            pl.BlockSpec((1, ba.shape[0]), lambda i: (0, 0)),
            pl.BlockSpec(wb.shape, lambda i: (0, 0)),
            pl.BlockSpec((1, bb.shape[0]), lambda i: (0, 0)),
        ],
        out_specs=pl.BlockSpec((bm, ko), lambda i: (i, 0)),
        out_shape=jax.ShapeDtypeStruct((m, ko), jnp.float32),
    )(x, wa, ba.reshape(1, -1), wb, bb.reshape(1, -1))


def _update_body(p0_ref, p1_ref, cnt_ref, ns_ref, w1a_ref, b1a_ref, w1b_ref,
                 b1b_ref, wu1_ref, bu1_ref, wu2_ref, bu2_ref, o_ref):
    ns = ns_ref[...]
    h1 = lax.dot_general(ns, w1a_ref[...], (((1,), (1,)), ((), ())),
                         preferred_element_type=jnp.float32)
    h1 = jnp.maximum(h1 + b1a_ref[...], 0.0)
    m1 = lax.dot_general(h1, w1b_ref[...], (((1,), (1,)), ((), ())),
                         preferred_element_type=jnp.float32)
    m1 = m1 + b1b_ref[...]
    sm = p0_ref[...] + p1_ref[...] + cnt_ref[...] * m1
    wu1 = wu1_ref[...]
    h = lax.dot_general(sm, wu1[:, :H], (((1,), (1,)), ((), ())),
                        preferred_element_type=jnp.float32)
    h = h + lax.dot_general(ns, wu1[:, H:], (((1,), (1,)), ((), ())),
                            preferred_element_type=jnp.float32)
    h = jnp.maximum(h + bu1_ref[...], 0.0)
    o = lax.dot_general(h, wu2_ref[...], (((1,), (1,)), ((), ())),
                        preferred_element_type=jnp.float32)
    o_ref[...] = o + bu2_ref[...]


def _tc_update(p0, p1, cnt, ns, w1a, b1a, w1b, b1b, wu1, bu1, wu2, bu2):
    bm = 1000
    row = lambda i: (i, 0)
    fix = lambda i: (0, 0)
    return pl.pallas_call(
        _update_body,
        grid=(N // bm,),
        in_specs=[
            pl.BlockSpec((bm, H), row),
            pl.BlockSpec((bm, H), row),
            pl.BlockSpec((bm, 1), row),
            pl.BlockSpec((bm, H), row),
            pl.BlockSpec(w1a.shape, fix),
            pl.BlockSpec((1, H), fix),
            pl.BlockSpec(w1b.shape, fix),
            pl.BlockSpec((1, H), fix),
            pl.BlockSpec(wu1.shape, fix),
            pl.BlockSpec((1, 2 * H), fix),
            pl.BlockSpec(wu2.shape, fix),
            pl.BlockSpec((1, H), fix),
        ],
        out_specs=pl.BlockSpec((bm, H), row),
        out_shape=jax.ShapeDtypeStruct((N, H), jnp.float32),
    )(p0, p1, cnt, ns, w1a, b1a.reshape(1, -1), w1b, b1b.reshape(1, -1),
      wu1, bu1.reshape(1, -1), wu2, bu2.reshape(1, -1))


def kernel(node_states, rel0, rel1, W0a, b0a, W0b, b0b, W1a, b1a, W1b, b1b,
           Wu1, bu1, Wu2, bu2):
    pairs = rel0.astype(jnp.int32).reshape(-1, 2)
    idxa = pairs[:, 0].reshape(-1, CH)
    idxb = pairs[:, 1].reshape(-1, CH)
    idx1 = rel1.astype(jnp.int32).reshape(-1, CH)
    lane = jnp.arange(CH, dtype=jnp.int32)
    gpad = (lane * 64) % N
    spad = N + lane
    idxa_g = _pad_chunks(idxa, NW * 8, gpad)
    idxb_g = _pad_chunks(idxb, NW * 8, gpad)
    idxa_s = _pad_chunks(idxa, NW * 8, spad)
    idxb_s = _pad_chunks(idxb, NW * 8, spad)
    idx1_s = _pad_chunks(idx1, NW * 8, spad)

    inp0 = _sc_gather(node_states, idxa_g, idxb_g)
    out0 = _tc_mlp(inp0, W0a, b0a, W0b, b0b, bm=2560)

    zeros = jnp.zeros((NPAD, H), jnp.float32)
    zcnt = jnp.zeros((NPAD,), jnp.int32)
    partials, cnts = _sc_scatter(out0, idxa_s, idxb_s, idx1_s, zeros, zcnt)

    cnt = (cnts[0, :N] + cnts[1, :N]).astype(jnp.float32).reshape(N, 1)
    return _tc_update(partials[0, :N], partials[1, :N], cnt, node_states,
                      W1a, b1a, W1b, b1b, Wu1, bu1, Wu2, bu2)


# speedup vs baseline: 7.0444x; 1.2715x over previous
import jax
import jax.numpy as jnp
from jax import lax
from jax.experimental import pallas as pl
from jax.experimental.pallas import tpu as pltpu
from jax.experimental.pallas import tpu_sc as plsc

N = 10000
NPAD = 10240
H = 128
CH = 128
NW = 32
NS = 16

_mesh = plsc.VectorSubcoreMesh(core_axis_name="c", subcore_axis_name="s")


def _pad_chunks(idx, mult, pad_row):
    pad = (-idx.shape[0]) % mult
    if pad:
        idx = jnp.concatenate([idx, jnp.broadcast_to(pad_row, (pad, CH))])
    return idx


def _sc_gather(table, idxa, idxb):
    nloc = idxa.shape[0] // NW

    def body(table_hbm, idxa_hbm, idxb_hbm, out_hbm,
             idxva, idxvb, bufa, bufb, sa0, sa1, sb0, sb1):
        w = lax.axis_index("s") * 2 + lax.axis_index("c")
        pltpu.sync_copy(idxa_hbm.at[pl.ds(w * nloc, nloc)], idxva)
        pltpu.sync_copy(idxb_hbm.at[pl.ds(w * nloc, nloc)], idxvb)
        sems_a = (sa0, sa1)
        sems_b = (sb0, sb1)
        pend = [None, None]
        for j in range(nloc + 1):
            p = j & 1
            if j < nloc:
                da = pltpu.async_copy(table_hbm.at[idxva.at[j]],
                                      bufa.at[p], sems_a[p])
                db = pltpu.async_copy(table_hbm.at[idxvb.at[j]],
                                      bufb.at[p], sems_b[p])
            if j >= 1:
                q = (j - 1) & 1
                chunk = w * nloc + (j - 1)
                pa, pb = pend[q]
                pa.wait()
                pltpu.sync_copy(bufa.at[q],
                                out_hbm.at[pl.ds(chunk * CH, CH), pl.ds(0, H)])
                pb.wait()
                pltpu.sync_copy(bufb.at[q],
                                out_hbm.at[pl.ds(chunk * CH, CH), pl.ds(H, H)])
            if j < nloc:
                pend[p] = (da, db)

    f = pl.kernel(
        body,
        out_type=jax.ShapeDtypeStruct((idxa.shape[0] * CH, 2 * H),
                                      jnp.float32),
        mesh=_mesh,
        scratch_types=[pltpu.VMEM((nloc, CH), jnp.int32),
                       pltpu.VMEM((nloc, CH), jnp.int32),
                       pltpu.VMEM((2, CH, H), jnp.float32),
                       pltpu.VMEM((2, CH, H), jnp.float32),
                       pltpu.SemaphoreType.DMA,
                       pltpu.SemaphoreType.DMA,
                       pltpu.SemaphoreType.DMA,
                       pltpu.SemaphoreType.DMA],
    )
    return f(table, idxa, idxb)


def _sc_scatter(rows0, idxa, idxb, idx1, zeros_hbm, zc_hbm):
    n0 = idxa.shape[0] // NW
    n1 = idx1.shape[0] // NW
    slab = NPAD // NS

    def body(rows0_hbm, idxa_hbm, idxb_hbm, idx1_hbm, z_hbm, zc, out_hbm,
             cnt_hbm, ga, gb, idxv1, buf, onesv, acc, acc_cnt, s0, s1):
        c = lax.axis_index("c")
        s = lax.axis_index("s")
        w = s * 2 + c
        pltpu.sync_copy(z_hbm.at[pl.ds(s * slab, slab)],
                        acc.at[pl.ds(s * slab, slab)])
        pltpu.sync_copy(zc.at[pl.ds(s * slab, slab)],
                        acc_cnt.at[pl.ds(s * slab, slab)])
        ones = jnp.ones((16,), jnp.int32)
        for k in range(CH // 16):
            onesv[pl.ds(k * 16, 16)] = ones
        plsc.subcore_barrier()

        pltpu.sync_copy(idx1_hbm.at[pl.ds(w * n1, n1)], idxv1)

        sems = (s0, s1)
        T = 2 * n0
        pend = [None, None]
        for t in range(T + 1):
            p = t & 1
            if t < T:
                j, st = t >> 1, t & 1
                if st == 0 and j % 8 == 0:
                    g = j // 8
                    gp = g & 1
                    pltpu.sync_copy(
                        idxa_hbm.at[pl.ds(w * n0 + g * 8, 8)], ga.at[gp])
                    pltpu.sync_copy(
                        idxb_hbm.at[pl.ds(w * n0 + g * 8, 8)], gb.at[gp])
                chunk = w * n0 + j
                cols = pl.ds(0, H) if st == 0 else pl.ds(H, H)
                r = pltpu.async_copy(rows0_hbm.at[pl.ds(chunk * CH, CH), cols],
                                     buf.at[p], sems[p])
            if t >= 1:
                q = (t - 1) & 1
                j1, st1 = (t - 1) >> 1, (t - 1) & 1
                gp1 = (j1 // 8) & 1
                idxref = (ga if st1 == 0 else gb).at[gp1, j1 % 8]
                pend[q].wait()
                pltpu.sync_copy(buf.at[q], acc.at[idxref], add=True)
            if t < T:
                pend[p] = r

        def it1(j, carry):
            pltpu.sync_copy(onesv, acc_cnt.at[idxv1.at[j]], add=True)
            return carry

        lax.fori_loop(0, n1, it1, 0)

        plsc.subcore_barrier()
        pltpu.sync_copy(acc.at[pl.ds(s * slab, slab)],
                        out_hbm.at[c, pl.ds(s * slab, slab)])
        pltpu.sync_copy(acc_cnt.at[pl.ds(s * slab, slab)],
                        cnt_hbm.at[c, pl.ds(s * slab, slab)])

    f = pl.kernel(
        body,
        out_type=(jax.ShapeDtypeStruct((2, NPAD, H), jnp.float32),
                  jax.ShapeDtypeStruct((2, NPAD), jnp.int32)),
        mesh=_mesh,
        scratch_types=[pltpu.VMEM((2, 8, CH), jnp.int32),
                       pltpu.VMEM((2, 8, CH), jnp.int32),
                       pltpu.VMEM((n1, CH), jnp.int32),
                       pltpu.VMEM((2, CH, H), jnp.float32),
                       pltpu.VMEM((CH,), jnp.int32),
                       pltpu.VMEM_SHARED((NPAD, H), jnp.float32),
                       pltpu.VMEM_SHARED((NPAD,), jnp.int32),
                       pltpu.SemaphoreType.DMA,
                       pltpu.SemaphoreType.DMA],
    )
    return f(rows0, idxa, idxb, idx1, zeros_hbm, zc_hbm)


def _mlp_body(x_ref, wa_ref, ba_ref, wb_ref, bb_ref, o_ref):
    x = x_ref[...].astype(jnp.bfloat16)
    wa = wa_ref[...].astype(jnp.bfloat16)
    h = lax.dot_general(x, wa, (((1,), (1,)), ((), ())),
                        preferred_element_type=jnp.float32)
    h = jnp.maximum(h + ba_ref[...], 0.0).astype(jnp.bfloat16)
    wb = wb_ref[...].astype(jnp.bfloat16)
    o = lax.dot_general(h, wb, (((1,), (1,)), ((), ())),
                        preferred_element_type=jnp.float32)
    o_ref[...] = o + bb_ref[...]


def _tc_mlp(x, wa, ba, wb, bb, bm):
    m, k = x.shape
    ko = wb.shape[0]
    return pl.pallas_call(
        _mlp_body,
        grid=(m // bm,),
        in_specs=[
            pl.BlockSpec((bm, k), lambda i: (i, 0)),
            pl.BlockSpec(wa.shape, lambda i: (0, 0)),
            pl.BlockSpec((1, ba.shape[0]), lambda i: (0, 0)),
            pl.BlockSpec(wb.shape, lambda i: (0, 0)),
            pl.BlockSpec((1, bb.shape[0]), lambda i: (0, 0)),
        ],
        out_specs=pl.BlockSpec((bm, ko), lambda i: (i, 0)),
        out_shape=jax.ShapeDtypeStruct((m, ko), jnp.float32),
    )(x, wa, ba.reshape(1, -1), wb, bb.reshape(1, -1))


def _update_body(p0_ref, p1_ref, cnt_ref, ns_ref, w1a_ref, b1a_ref, w1b_ref,
                 b1b_ref, wu1_ref, bu1_ref, wu2_ref, bu2_ref, o_ref):
    ns = ns_ref[...]
    h1 = lax.dot_general(ns, w1a_ref[...], (((1,), (1,)), ((), ())),
                         preferred_element_type=jnp.float32)
    h1 = jnp.maximum(h1 + b1a_ref[...], 0.0)
    m1 = lax.dot_general(h1, w1b_ref[...], (((1,), (1,)), ((), ())),
                         preferred_element_type=jnp.float32)
    m1 = m1 + b1b_ref[...]
    sm = p0_ref[...] + p1_ref[...] + cnt_ref[...] * m1
    wu1 = wu1_ref[...]
    h = lax.dot_general(sm, wu1[:, :H], (((1,), (1,)), ((), ())),
                        preferred_element_type=jnp.float32)
    h = h + lax.dot_general(ns, wu1[:, H:], (((1,), (1,)), ((), ())),
                            preferred_element_type=jnp.float32)
    h = jnp.maximum(h + bu1_ref[...], 0.0)
    o = lax.dot_general(h, wu2_ref[...], (((1,), (1,)), ((), ())),
                        preferred_element_type=jnp.float32)
    o_ref[...] = o + bu2_ref[...]


def _tc_update(p0, p1, cnt, ns, w1a, b1a, w1b, b1b, wu1, bu1, wu2, bu2):
    bm = 1000
    row = lambda i: (i, 0)
    fix = lambda i: (0, 0)
    return pl.pallas_call(
        _update_body,
        grid=(N // bm,),
        in_specs=[
            pl.BlockSpec((bm, H), row),
            pl.BlockSpec((bm, H), row),
            pl.BlockSpec((bm, 1), row),
            pl.BlockSpec((bm, H), row),
            pl.BlockSpec(w1a.shape, fix),
            pl.BlockSpec((1, H), fix),
            pl.BlockSpec(w1b.shape, fix),
            pl.BlockSpec((1, H), fix),
            pl.BlockSpec(wu1.shape, fix),
            pl.BlockSpec((1, 2 * H), fix),
            pl.BlockSpec(wu2.shape, fix),
            pl.BlockSpec((1, H), fix),
        ],
        out_specs=pl.BlockSpec((bm, H), row),
        out_shape=jax.ShapeDtypeStruct((N, H), jnp.float32),
    )(p0, p1, cnt, ns, w1a, b1a.reshape(1, -1), w1b, b1b.reshape(1, -1),
      wu1, bu1.reshape(1, -1), wu2, bu2.reshape(1, -1))


def kernel(node_states, rel0, rel1, W0a, b0a, W0b, b0b, W1a, b1a, W1b, b1b,
           Wu1, bu1, Wu2, bu2):
    r0 = rel0.astype(jnp.int32)
    idxa = r0[0::2].reshape(-1, CH)
    idxb = r0[1::2].reshape(-1, CH)
    idx1 = rel1.astype(jnp.int32).reshape(-1, CH)
    lane = jnp.arange(CH, dtype=jnp.int32)
    gpad = (lane * 64) % N
    spad = N + lane
    idxa_g = _pad_chunks(idxa, NW * 8, gpad)
    idxb_g = _pad_chunks(idxb, NW * 8, gpad)
    idxa_s = _pad_chunks(idxa, NW * 8, spad)
    idxb_s = _pad_chunks(idxb, NW * 8, spad)
    idx1_s = _pad_chunks(idx1, NW * 8, spad)

    inp0 = _sc_gather(node_states, idxa_g, idxb_g)
    out0 = _tc_mlp(inp0, W0a, b0a, W0b, b0b, bm=4096)

    zeros = jnp.zeros((NPAD, H), jnp.float32)
    zcnt = jnp.zeros((NPAD,), jnp.int32)
    partials, cnts = _sc_scatter(out0, idxa_s, idxb_s, idx1_s, zeros, zcnt)

    cnt = (cnts[0, :N] + cnts[1, :N]).astype(jnp.float32).reshape(N, 1)
    return _tc_update(partials[0, :N], partials[1, :N], cnt, node_states,
                      W1a, b1a, W1b, b1b, Wu1, bu1, Wu2, bu2)
